# baseline (speedup 1.0000x reference)
import jax
import jax.numpy as jnp
from jax import lax
from jax.experimental import pallas as pl
from jax.experimental.pallas import tpu as pltpu

_N = 4194304
_HEAD = 1024
_CHUNKS = 8
_CHUNK = _N // _CHUNKS


def _body(c_ref, w_ref, o_ref, scratch, head_sem, *sems):
    copies = []
    for i in range(_CHUNKS):
        lo = _HEAD if i == 0 else i * _CHUNK
        sz = (i + 1) * _CHUNK - lo
        cp = pltpu.make_async_copy(
            w_ref.at[pl.ds(lo, sz)], o_ref.at[pl.ds(lo, sz)], sems[i]
        )
        cp.start()
        copies.append(cp)

    head_in = pltpu.make_async_copy(
        w_ref.at[pl.ds(0, _HEAD)], scratch, head_sem
    )
    head_in.start()
    head_in.wait()

    a = c_ref[9] / (c_ref[10] * 400000.0) * 0.001
    b = c_ref[11] / c_ref[10]
    blk = scratch[...]
    idx = lax.broadcasted_iota(jnp.int32, blk.shape, 0)
    blk = jnp.where(idx == 0, a, blk)
    blk = jnp.where(idx == 1, b, blk)
    scratch[...] = blk

    head_out = pltpu.make_async_copy(
        scratch, o_ref.at[pl.ds(0, _HEAD)], head_sem
    )
    head_out.start()
    head_out.wait()
    for cp in copies:
        cp.wait()


def kernel(y, w, c, t):
    return pl.pallas_call(
        _body,
        in_specs=[
            pl.BlockSpec(memory_space=pltpu.SMEM),
            pl.BlockSpec(memory_space=pl.ANY),
        ],
        out_specs=pl.BlockSpec(memory_space=pl.ANY),
        out_shape=jax.ShapeDtypeStruct((_N,), jnp.float32),
        scratch_shapes=[pltpu.VMEM((_HEAD,), jnp.float32)]
        + [pltpu.SemaphoreType.DMA] * (1 + _CHUNKS),
    )(c, w)


# speedup vs baseline: 28.7918x; 28.7918x over previous
import jax
import jax.numpy as jnp
from jax import lax
from jax.experimental import pallas as pl
from jax.experimental.pallas import tpu as pltpu

_N = 4194304
_GRID = 16
_BLOCK = _N // _GRID


def _body(c_ref, w_ref, o_ref):
    o_ref[...] = w_ref[...]

    @pl.when(pl.program_id(0) == 0)
    def _patch():
        a = c_ref[9] / (c_ref[10] * 400000.0) * 0.001
        b = c_ref[11] / c_ref[10]
        head = w_ref[pl.ds(0, 128)]
        idx = lax.broadcasted_iota(jnp.int32, head.shape, 0)
        head = jnp.where(idx == 0, a, head)
        head = jnp.where(idx == 1, b, head)
        o_ref[pl.ds(0, 128)] = head


def kernel(y, w, c, t):
    return pl.pallas_call(
        _body,
        grid=(_GRID,),
        in_specs=[
            pl.BlockSpec(memory_space=pltpu.SMEM),
            pl.BlockSpec((_BLOCK,), lambda i: (i,)),
        ],
        out_specs=pl.BlockSpec((_BLOCK,), lambda i: (i,)),
        out_shape=jax.ShapeDtypeStruct((_N,), jnp.float32),
    )(c, w)
